# Initial kernel scaffold; baseline (speedup 1.0000x reference)
import functools

import jax
import jax.numpy as jnp
from jax import lax
from jax.experimental import pallas as pl
from jax.experimental.pallas import tpu as pltpu
from jax.experimental.pallas import tpu_sc as plsc

K = 1024
D = 1024
DDEC = 256
B = 16384

NC, NS = 2, 16
NW = NC * NS
B_PER_W = B // NW
CH = 128
NCH = B_PER_W // CH


def _table_body(emb_ref, w_ref, b_ref, out_ref):
    acc = jnp.dot(emb_ref[...], w_ref[...],
                  preferred_element_type=jnp.float32,
                  precision=lax.Precision.HIGHEST)
    out_ref[...] = jnp.tanh(acc + b_ref[...])


def _build_table(embedding, W_dec, b_dec):
    return pl.pallas_call(
        _table_body,
        out_shape=jax.ShapeDtypeStruct((K, DDEC), jnp.float32),
    )(embedding, W_dec, b_dec.reshape(1, DDEC))


def _gather_body(table_hbm, idx_hbm, out_hbm, idx_v, rows0, rows1, sem0, sem1,
                 osem0, osem1):
    wid = lax.axis_index("s") * NC + lax.axis_index("c")
    base = wid * B_PER_W
    pltpu.sync_copy(idx_hbm.at[pl.ds(base, B_PER_W)], idx_v)

    rows = (rows0, rows1)
    gsem = (sem0, sem1)
    osem = (osem0, osem1)

    pltpu.async_copy(table_hbm.at[idx_v.at[pl.ds(0, CH)]], rows0, sem0)
    for ci in range(NCH):
        cur = ci % 2
        nxt = (ci + 1) % 2
        if ci + 1 < NCH:
            pltpu.async_copy(
                table_hbm.at[idx_v.at[pl.ds((ci + 1) * CH, CH)]],
                rows[nxt], gsem[nxt])
        pltpu.make_async_copy(table_hbm.at[idx_v.at[pl.ds(ci * CH, CH)]],
                              rows[cur], gsem[cur]).wait()
        if ci >= 2:
            pltpu.make_async_copy(
                rows[cur], out_hbm.at[pl.ds(base + (ci - 2) * CH, CH)],
                osem[cur]).wait()
        pltpu.async_copy(rows[cur], out_hbm.at[pl.ds(base + ci * CH, CH)],
                         osem[cur])
    for ci in range(max(0, NCH - 2), NCH):
        cur = ci % 2
        pltpu.make_async_copy(rows[cur],
                              out_hbm.at[pl.ds(base + ci * CH, CH)],
                              osem[cur]).wait()


@functools.partial(
    pl.kernel,
    mesh=plsc.VectorSubcoreMesh(core_axis_name="c", subcore_axis_name="s"),
    out_type=jax.ShapeDtypeStruct((B, DDEC), jnp.float32),
    scratch_types=[
        pltpu.VMEM((B_PER_W,), jnp.int32),
        pltpu.VMEM((CH, DDEC), jnp.float32),
        pltpu.VMEM((CH, DDEC), jnp.float32),
        pltpu.SemaphoreType.DMA,
        pltpu.SemaphoreType.DMA,
        pltpu.SemaphoreType.DMA,
        pltpu.SemaphoreType.DMA,
    ],
)
def _gather_rows(table_hbm, idx_hbm, out_hbm, idx_v, rows0, rows1, sem0, sem1,
                 osem0, osem1):
    _gather_body(table_hbm, idx_hbm, out_hbm, idx_v, rows0, rows1, sem0, sem1,
                 osem0, osem1)


def kernel(encoding_indices, embedding, W_dec, b_dec):
    bs, m = encoding_indices.shape
    table = _build_table(embedding, W_dec, b_dec)
    flat_idx = encoding_indices.reshape(-1)
    out = _gather_rows(table, flat_idx)
    return out.reshape(bs, m, DDEC)


# speedup vs baseline: 6.5070x; 6.5070x over previous
import functools

import jax
import jax.numpy as jnp
from jax import lax
from jax.experimental import pallas as pl
from jax.experimental.pallas import tpu as pltpu
from jax.experimental.pallas import tpu_sc as plsc

K = 1024
D = 1024
DDEC = 256
B = 16384

NC, NS = 2, 16
NW = NC * NS
B_PER_W = B // NW
CH = 128
NCH = B_PER_W // CH


def _table_body(emb_ref, w_ref, b_ref, out_ref):
    acc = jnp.dot(emb_ref[...], w_ref[...],
                  preferred_element_type=jnp.float32,
                  precision=lax.Precision.HIGHEST)
    out_ref[...] = jnp.tanh(acc + b_ref[...])


def _build_table(embedding, W_dec, b_dec):
    return pl.pallas_call(
        _table_body,
        out_shape=jax.ShapeDtypeStruct((K, DDEC), jnp.float32),
    )(embedding, W_dec, b_dec.reshape(1, DDEC))


def _gather_body(table_hbm, idx_hbm, out_hbm, idx_v, rows0, rows1, sem0, sem1,
                 osem0, osem1):
    wid = lax.axis_index("s") * NC + lax.axis_index("c")
    base = wid * B_PER_W
    pltpu.sync_copy(idx_hbm.at[pl.ds(base, B_PER_W)], idx_v)

    rows = (rows0, rows1)
    gsem = (sem0, sem1)
    osem = (osem0, osem1)

    def fire_gather(ci, buf):
        pltpu.async_copy(table_hbm.at[idx_v.at[pl.ds(ci * CH, CH)]],
                         rows[buf], gsem[buf])

    def wait_gather(ci, buf):
        pltpu.make_async_copy(table_hbm.at[idx_v.at[pl.ds(ci * CH, CH)]],
                              rows[buf], gsem[buf]).wait()

    def fire_store(ci, buf):
        pltpu.async_copy(rows[buf], out_hbm.at[pl.ds(base + ci * CH, CH)],
                         osem[buf])

    def wait_store(ci, buf):
        pltpu.make_async_copy(rows[buf], out_hbm.at[pl.ds(base + ci * CH, CH)],
                              osem[buf]).wait()

    fire_gather(0, 0)
    fire_gather(1, 1)
    for ci in range(NCH):
        cur = ci % 2
        wait_gather(ci, cur)
        fire_store(ci, cur)
        if ci + 2 < NCH:
            wait_store(ci, cur)
            fire_gather(ci + 2, cur)
    for ci in range(NCH - 2, NCH):
        wait_store(ci, ci % 2)


@functools.partial(
    pl.kernel,
    mesh=plsc.VectorSubcoreMesh(core_axis_name="c", subcore_axis_name="s"),
    out_type=jax.ShapeDtypeStruct((B, DDEC), jnp.float32),
    scratch_types=[
        pltpu.VMEM((B_PER_W,), jnp.int32),
        pltpu.VMEM((CH, DDEC), jnp.float32),
        pltpu.VMEM((CH, DDEC), jnp.float32),
        pltpu.SemaphoreType.DMA,
        pltpu.SemaphoreType.DMA,
        pltpu.SemaphoreType.DMA,
        pltpu.SemaphoreType.DMA,
    ],
)
def _gather_rows(table_hbm, idx_hbm, out_hbm, idx_v, rows0, rows1, sem0, sem1,
                 osem0, osem1):
    _gather_body(table_hbm, idx_hbm, out_hbm, idx_v, rows0, rows1, sem0, sem1,
                 osem0, osem1)


def kernel(encoding_indices, embedding, W_dec, b_dec):
    bs, m = encoding_indices.shape
    table = _build_table(embedding, W_dec, b_dec)
    flat_idx = encoding_indices.reshape(-1)
    out = _gather_rows(table, flat_idx)
    return out.reshape(bs, m, DDEC)


# speedup vs baseline: 7.0736x; 1.0871x over previous
import functools

import jax
import jax.numpy as jnp
from jax import lax
from jax.experimental import pallas as pl
from jax.experimental.pallas import tpu as pltpu
from jax.experimental.pallas import tpu_sc as plsc

K = 1024
D = 1024
DDEC = 256
B = 16384

NC, NS = 2, 16
NW = NC * NS
B_PER_W = B // NW
CH = 128
NCH = B_PER_W // CH


def _table_body(emb_ref, w_ref, b_ref, out_ref):
    acc = jnp.dot(emb_ref[...], w_ref[...],
                  preferred_element_type=jnp.float32)
    out_ref[...] = jnp.tanh(acc + b_ref[...])


def _build_table(embedding, W_dec, b_dec):
    return pl.pallas_call(
        _table_body,
        out_shape=jax.ShapeDtypeStruct((K, DDEC), jnp.float32),
    )(embedding, W_dec, b_dec.reshape(1, DDEC))


def _gather_body(table_hbm, idx_hbm, out_hbm, idx_v, rows0, rows1, sem0, sem1,
                 osem0, osem1):
    wid = lax.axis_index("s") * NC + lax.axis_index("c")
    base = wid * B_PER_W
    pltpu.sync_copy(idx_hbm.at[pl.ds(base, B_PER_W)], idx_v)

    rows = (rows0, rows1)
    gsem = (sem0, sem1)
    osem = (osem0, osem1)

    def fire_gather(ci, buf):
        pltpu.async_copy(table_hbm.at[idx_v.at[pl.ds(ci * CH, CH)]],
                         rows[buf], gsem[buf])

    def wait_gather(ci, buf):
        pltpu.make_async_copy(table_hbm.at[idx_v.at[pl.ds(ci * CH, CH)]],
                              rows[buf], gsem[buf]).wait()

    def fire_store(ci, buf):
        pltpu.async_copy(rows[buf], out_hbm.at[pl.ds(base + ci * CH, CH)],
                         osem[buf])

    def wait_store(ci, buf):
        pltpu.make_async_copy(rows[buf], out_hbm.at[pl.ds(base + ci * CH, CH)],
                              osem[buf]).wait()

    fire_gather(0, 0)
    fire_gather(1, 1)
    for ci in range(NCH):
        cur = ci % 2
        wait_gather(ci, cur)
        fire_store(ci, cur)
        if ci + 2 < NCH:
            wait_store(ci, cur)
            fire_gather(ci + 2, cur)
    for ci in range(NCH - 2, NCH):
        wait_store(ci, ci % 2)


@functools.partial(
    pl.kernel,
    mesh=plsc.VectorSubcoreMesh(core_axis_name="c", subcore_axis_name="s"),
    out_type=jax.ShapeDtypeStruct((B, DDEC), jnp.float32),
    scratch_types=[
        pltpu.VMEM((B_PER_W,), jnp.int32),
        pltpu.VMEM((CH, DDEC), jnp.float32),
        pltpu.VMEM((CH, DDEC), jnp.float32),
        pltpu.SemaphoreType.DMA,
        pltpu.SemaphoreType.DMA,
        pltpu.SemaphoreType.DMA,
        pltpu.SemaphoreType.DMA,
    ],
)
def _gather_rows(table_hbm, idx_hbm, out_hbm, idx_v, rows0, rows1, sem0, sem1,
                 osem0, osem1):
    _gather_body(table_hbm, idx_hbm, out_hbm, idx_v, rows0, rows1, sem0, sem1,
                 osem0, osem1)


def kernel(encoding_indices, embedding, W_dec, b_dec):
    bs, m = encoding_indices.shape
    table = _build_table(embedding, W_dec, b_dec)
    flat_idx = encoding_indices.reshape(-1)
    out = _gather_rows(table, flat_idx)
    return out.reshape(bs, m, DDEC)


# speedup vs baseline: 7.0887x; 1.0021x over previous
import functools

import jax
import jax.numpy as jnp
from jax import lax
from jax.experimental import pallas as pl
from jax.experimental.pallas import tpu as pltpu
from jax.experimental.pallas import tpu_sc as plsc

K = 1024
D = 1024
DDEC = 256
B = 16384

NC, NS = 2, 16
NW = NC * NS
B_PER_W = B // NW
CH = 128
NCH = B_PER_W // CH
NBUF = 3


def _table_body(emb_ref, w_ref, b_ref, out_ref):
    acc = jnp.dot(emb_ref[...], w_ref[...],
                  preferred_element_type=jnp.float32)
    out_ref[...] = jnp.tanh(acc + b_ref[...])


def _build_table(embedding, W_dec, b_dec):
    return pl.pallas_call(
        _table_body,
        out_shape=jax.ShapeDtypeStruct((K, DDEC), jnp.float32),
    )(embedding, W_dec, b_dec.reshape(1, DDEC))


def _gather_body(table_hbm, idx_hbm, out_hbm, idx_v, rows, gsem, osem):
    wid = lax.axis_index("s") * NC + lax.axis_index("c")
    base = wid * B_PER_W
    pltpu.sync_copy(idx_hbm.at[pl.ds(base, B_PER_W)], idx_v)

    def fire_gather(ci, buf):
        pltpu.async_copy(table_hbm.at[idx_v.at[pl.ds(ci * CH, CH)]],
                         rows[buf], gsem[buf])

    def wait_gather(ci, buf):
        pltpu.make_async_copy(table_hbm.at[idx_v.at[pl.ds(ci * CH, CH)]],
                              rows[buf], gsem[buf]).wait()

    def fire_store(ci, buf):
        pltpu.async_copy(rows[buf], out_hbm.at[pl.ds(base + ci * CH, CH)],
                         osem[buf])

    def wait_store(ci, buf):
        pltpu.make_async_copy(rows[buf], out_hbm.at[pl.ds(base + ci * CH, CH)],
                              osem[buf]).wait()

    for ci in range(min(NBUF, NCH)):
        fire_gather(ci, ci % NBUF)
    for ci in range(NCH):
        buf = ci % NBUF
        wait_gather(ci, buf)
        fire_store(ci, buf)
        if ci + NBUF < NCH:
            wait_store(ci, buf)
            fire_gather(ci + NBUF, buf)
    for ci in range(max(0, NCH - NBUF), NCH):
        wait_store(ci, ci % NBUF)


@functools.partial(
    pl.kernel,
    mesh=plsc.VectorSubcoreMesh(core_axis_name="c", subcore_axis_name="s"),
    out_type=jax.ShapeDtypeStruct((B, DDEC), jnp.float32),
    scratch_types=[
        pltpu.VMEM((B_PER_W,), jnp.int32),
        pltpu.VMEM((CH, DDEC), jnp.float32),
        pltpu.VMEM((CH, DDEC), jnp.float32),
        pltpu.VMEM((CH, DDEC), jnp.float32),
        pltpu.SemaphoreType.DMA,
        pltpu.SemaphoreType.DMA,
        pltpu.SemaphoreType.DMA,
        pltpu.SemaphoreType.DMA,
        pltpu.SemaphoreType.DMA,
        pltpu.SemaphoreType.DMA,
    ],
)
def _gather_rows(table_hbm, idx_hbm, out_hbm, idx_v, rows0, rows1, rows2,
                 g0, g1, g2, o0, o1, o2):
    _gather_body(table_hbm, idx_hbm, out_hbm, idx_v,
                 (rows0, rows1, rows2), (g0, g1, g2), (o0, o1, o2))


def kernel(encoding_indices, embedding, W_dec, b_dec):
    bs, m = encoding_indices.shape
    table = _build_table(embedding, W_dec, b_dec)
    flat_idx = encoding_indices.reshape(-1)
    out = _gather_rows(table, flat_idx)
    return out.reshape(bs, m, DDEC)


# speedup vs baseline: 7.1260x; 1.0053x over previous
import functools

import jax
import jax.numpy as jnp
from jax import lax
from jax.experimental import pallas as pl
from jax.experimental.pallas import tpu as pltpu
from jax.experimental.pallas import tpu_sc as plsc

K = 1024
D = 1024
DDEC = 256
B = 16384

NC, NS = 2, 16
NW = NC * NS
B_PER_W = B // NW
CH = 128
NCH = B_PER_W // CH
NBUF = 3


def _table_body(emb_ref, w_ref, b_ref, out_ref):
    acc = jnp.dot(emb_ref[...].astype(jnp.bfloat16),
                  w_ref[...].astype(jnp.bfloat16),
                  preferred_element_type=jnp.float32)
    out_ref[...] = jnp.tanh(acc + b_ref[...])


def _build_table(embedding, W_dec, b_dec):
    return pl.pallas_call(
        _table_body,
        out_shape=jax.ShapeDtypeStruct((K, DDEC), jnp.float32),
    )(embedding, W_dec, b_dec.reshape(1, DDEC))


def _gather_body(table_hbm, idx_hbm, out_hbm, idx_v, rows, gsem, osem):
    wid = lax.axis_index("s") * NC + lax.axis_index("c")
    base = wid * B_PER_W
    pltpu.sync_copy(idx_hbm.at[pl.ds(base, B_PER_W)], idx_v)

    def fire_gather(ci, buf):
        pltpu.async_copy(table_hbm.at[idx_v.at[pl.ds(ci * CH, CH)]],
                         rows[buf], gsem[buf])

    def wait_gather(ci, buf):
        pltpu.make_async_copy(table_hbm.at[idx_v.at[pl.ds(ci * CH, CH)]],
                              rows[buf], gsem[buf]).wait()

    def fire_store(ci, buf):
        pltpu.async_copy(rows[buf], out_hbm.at[pl.ds(base + ci * CH, CH)],
                         osem[buf])

    def wait_store(ci, buf):
        pltpu.make_async_copy(rows[buf], out_hbm.at[pl.ds(base + ci * CH, CH)],
                              osem[buf]).wait()

    for ci in range(min(NBUF, NCH)):
        fire_gather(ci, ci % NBUF)
    for ci in range(NCH):
        buf = ci % NBUF
        wait_gather(ci, buf)
        fire_store(ci, buf)
        if ci + NBUF < NCH:
            wait_store(ci, buf)
            fire_gather(ci + NBUF, buf)
    for ci in range(max(0, NCH - NBUF), NCH):
        wait_store(ci, ci % NBUF)


@functools.partial(
    pl.kernel,
    mesh=plsc.VectorSubcoreMesh(core_axis_name="c", subcore_axis_name="s"),
    out_type=jax.ShapeDtypeStruct((B, DDEC), jnp.float32),
    scratch_types=[
        pltpu.VMEM((B_PER_W,), jnp.int32),
        pltpu.VMEM((CH, DDEC), jnp.float32),
        pltpu.VMEM((CH, DDEC), jnp.float32),
        pltpu.VMEM((CH, DDEC), jnp.float32),
        pltpu.SemaphoreType.DMA,
        pltpu.SemaphoreType.DMA,
        pltpu.SemaphoreType.DMA,
        pltpu.SemaphoreType.DMA,
        pltpu.SemaphoreType.DMA,
        pltpu.SemaphoreType.DMA,
    ],
)
def _gather_rows(table_hbm, idx_hbm, out_hbm, idx_v, rows0, rows1, rows2,
                 g0, g1, g2, o0, o1, o2):
    _gather_body(table_hbm, idx_hbm, out_hbm, idx_v,
                 (rows0, rows1, rows2), (g0, g1, g2), (o0, o1, o2))


def kernel(encoding_indices, embedding, W_dec, b_dec):
    bs, m = encoding_indices.shape
    table = _build_table(embedding, W_dec, b_dec)
    flat_idx = encoding_indices.reshape(-1)
    out = _gather_rows(table, flat_idx)
    return out.reshape(bs, m, DDEC)
